# baseline (speedup 1.0000x reference)
import functools

import jax
import jax.numpy as jnp
from jax import lax
from jax.experimental import pallas as pl
from jax.experimental.pallas import tpu as pltpu
from jax.experimental.pallas import tpu_sc as plsc

EMBED_DIM = 64
NUM_CORES = 2
NUM_SUBCORES = 16
NUM_WORKERS = NUM_CORES * NUM_SUBCORES
CHUNK = 128


@functools.partial(jax.jit, static_argnums=())
def _embedding_lookup(idx_flat, table):
    total = idx_flat.shape[0]
    per_worker = total // NUM_WORKERS
    n_chunks = per_worker // CHUNK

    mesh = plsc.VectorSubcoreMesh(
        core_axis_name="c",
        subcore_axis_name="s",
        num_cores=NUM_CORES,
        num_subcores=NUM_SUBCORES,
    )

    @functools.partial(
        pl.kernel,
        mesh=mesh,
        compiler_params=pltpu.CompilerParams(use_tc_tiling_on_sc=False),
        out_type=jax.ShapeDtypeStruct((total, EMBED_DIM), jnp.float32),
        scratch_types=[
            pltpu.VMEM((per_worker,), jnp.int32),
            pltpu.VMEM((CHUNK, EMBED_DIM), jnp.float32),
            pltpu.SemaphoreType.DMA,
        ],
    )
    def emb(idx_hbm, table_hbm, out_hbm, idx_v, rows_v, sem):
        wid = lax.axis_index("s") * NUM_CORES + lax.axis_index("c")
        base = wid * per_worker
        pltpu.sync_copy(idx_hbm.at[pl.ds(base, per_worker)], idx_v)

        def step(j, carry):
            off = j * CHUNK
            pltpu.async_copy(
                table_hbm.at[idx_v.at[pl.ds(off, CHUNK)]], rows_v, sem
            ).wait()
            pltpu.sync_copy(rows_v, out_hbm.at[pl.ds(base + off, CHUNK)])
            return carry

        lax.fori_loop(0, n_chunks, step, 0)

    return emb(idx_flat, table)


def kernel(input_sentence, W):
    batch, seq = input_sentence.shape
    idx_flat = input_sentence.reshape(-1).astype(jnp.int32)
    out = _embedding_lookup(idx_flat, W)
    return out.reshape(batch, seq, EMBED_DIM)


# speedup vs baseline: 1.0430x; 1.0430x over previous
import functools

import jax
import jax.numpy as jnp
from jax import lax
from jax.experimental import pallas as pl
from jax.experimental.pallas import tpu as pltpu
from jax.experimental.pallas import tpu_sc as plsc

EMBED_DIM = 64
NUM_CORES = 2
NUM_SUBCORES = 16
NUM_WORKERS = NUM_CORES * NUM_SUBCORES
CHUNK = 128
NBUF = 5


@jax.jit
def _embedding_lookup(idx_flat, table):
    total = idx_flat.shape[0]
    per_worker = total // NUM_WORKERS
    n_chunks = per_worker // CHUNK
    n_groups = n_chunks // NBUF

    mesh = plsc.VectorSubcoreMesh(
        core_axis_name="c",
        subcore_axis_name="s",
        num_cores=NUM_CORES,
        num_subcores=NUM_SUBCORES,
    )

    @functools.partial(
        pl.kernel,
        mesh=mesh,
        compiler_params=pltpu.CompilerParams(use_tc_tiling_on_sc=False),
        out_type=jax.ShapeDtypeStruct((total, EMBED_DIM), jnp.float32),
        scratch_types=(
            [pltpu.VMEM((per_worker,), jnp.int32)]
            + [pltpu.VMEM((CHUNK, EMBED_DIM), jnp.float32) for _ in range(NBUF)]
            + [pltpu.SemaphoreType.DMA for _ in range(2 * NBUF)]
        ),
    )
    def emb(idx_hbm, table_hbm, out_hbm, idx_v, *rest):
        bufs = rest[:NBUF]
        sem_g = rest[NBUF : 2 * NBUF]
        sem_w = rest[2 * NBUF :]

        wid = lax.axis_index("s") * NUM_CORES + lax.axis_index("c")
        base = wid * per_worker
        pltpu.sync_copy(idx_hbm.at[pl.ds(base, per_worker)], idx_v)

        def gather(j, b):
            off = j * CHUNK
            return pltpu.make_async_copy(
                table_hbm.at[idx_v.at[pl.ds(off, CHUNK)]], bufs[b], sem_g[b]
            )

        def writeback(j, b):
            off = j * CHUNK
            return pltpu.make_async_copy(
                bufs[b], out_hbm.at[pl.ds(base + off, CHUNK)], sem_w[b]
            )

        for b in range(NBUF):
            gather(b, b).start()

        def group(g, carry):
            j0 = g * NBUF
            for b in range(NBUF):
                gather(j0 + b, b).wait()
                writeback(j0 + b, b).start()
            for b in range(NBUF):
                writeback(j0 + b, b).wait()
                gather(j0 + NBUF + b, b).start()
            return carry

        lax.fori_loop(0, n_groups - 1, group, 0)

        j0 = (n_groups - 1) * NBUF
        for b in range(NBUF):
            gather(j0 + b, b).wait()
            writeback(j0 + b, b).start()
        for b in range(NBUF):
            writeback(j0 + b, b).wait()

    return emb(idx_flat, table)


def kernel(input_sentence, W):
    batch, seq = input_sentence.shape
    idx_flat = input_sentence.reshape(-1).astype(jnp.int32)
    out = _embedding_lookup(idx_flat, W)
    return out.reshape(batch, seq, EMBED_DIM)


# speedup vs baseline: 1.0601x; 1.0164x over previous
import functools

import jax
import jax.numpy as jnp
from jax import lax
from jax.experimental import pallas as pl
from jax.experimental.pallas import tpu as pltpu
from jax.experimental.pallas import tpu_sc as plsc

EMBED_DIM = 64
NUM_CORES = 2
NUM_SUBCORES = 16
NUM_WORKERS = NUM_CORES * NUM_SUBCORES
CHUNK = 128
NBUF = 5


@jax.jit
def _embedding_lookup(idx_flat, table):
    total = idx_flat.shape[0]
    per_worker = total // NUM_WORKERS
    n_chunks = per_worker // CHUNK
    n_groups = n_chunks // NBUF

    mesh = plsc.VectorSubcoreMesh(
        core_axis_name="c",
        subcore_axis_name="s",
        num_cores=NUM_CORES,
        num_subcores=NUM_SUBCORES,
    )

    @functools.partial(
        pl.kernel,
        mesh=mesh,
        compiler_params=pltpu.CompilerParams(use_tc_tiling_on_sc=False),
        out_type=jax.ShapeDtypeStruct((total, EMBED_DIM), jnp.float32),
        scratch_types=(
            [pltpu.VMEM((per_worker,), jnp.int32)]
            + [pltpu.VMEM((CHUNK, EMBED_DIM), jnp.float32) for _ in range(NBUF)]
            + [pltpu.SemaphoreType.DMA for _ in range(2 * NBUF)]
        ),
    )
    def emb(idx_hbm, table_hbm, out_hbm, idx_v, *rest):
        bufs = rest[:NBUF]
        sem_g = rest[NBUF : 2 * NBUF]
        sem_w = rest[2 * NBUF :]

        wid = lax.axis_index("s") * NUM_CORES + lax.axis_index("c")
        base = wid * per_worker
        pltpu.sync_copy(idx_hbm.at[pl.ds(base, per_worker)], idx_v)

        def gather(j, b):
            off = j * CHUNK
            return pltpu.make_async_copy(
                table_hbm.at[idx_v.at[pl.ds(off, CHUNK)]], bufs[b], sem_g[b]
            )

        def writeback(j, b):
            off = j * CHUNK
            return pltpu.make_async_copy(
                bufs[b], out_hbm.at[pl.ds(base + off, CHUNK)], sem_w[b]
            )

        for b in range(NBUF):
            gather(b, b).start()

        def group(g, carry):
            j0 = g * NBUF
            for b in range(NBUF):
                gather(j0 + b, b).wait()
                writeback(j0 + b, b).start()
            for b in range(NBUF):
                writeback(j0 + b, b).wait()
                gather(j0 + NBUF + b, b).start()
            return carry

        lax.fori_loop(0, n_groups - 1, group, 0)

        j0 = (n_groups - 1) * NBUF
        for b in range(NBUF):
            gather(j0 + b, b).wait()
            writeback(j0 + b, b).start()
        for b in range(NBUF):
            writeback(j0 + b, b).wait()

    return emb(idx_flat, table)


def kernel(input_sentence, W):
    batch, seq = input_sentence.shape
    idx_flat = input_sentence.T.reshape(-1).astype(jnp.int32)
    out = _embedding_lookup(idx_flat, W)
    return out.reshape(seq, batch, EMBED_DIM).transpose(1, 0, 2)
